# baseline (speedup 1.0000x reference)
import functools

import jax
import jax.numpy as jnp
from jax import lax
from jax.experimental import pallas as pl
from jax.experimental.pallas import tpu as pltpu
from jax.experimental.pallas import tpu_sc as plsc

B, H, W = 64, 512, 512
HW = H * W
D = 2
P = 4096
ROWS = D * B
RP = 2 * P
NC, NS = 2, 16
NW = NC * NS
ROWS_PER_TILE = ROWS // NW
L = 16
CHUNK = 128
FIRE = 8


def _sc_body(xf, bidx, didx, out, bi_v, di_v, gi_v, vals_v, sem):
    cid = lax.axis_index("c")
    sid = lax.axis_index("s")
    wid = sid * NC + cid
    lane2 = 2 * lax.iota(jnp.int32, L)

    for j in range(ROWS_PER_TILE):
        r = wid * ROWS_PER_TILE + j
        off = lax.rem(r, B) * HW

        pltpu.sync_copy(bidx.at[r], bi_v)
        pltpu.sync_copy(didx.at[r], di_v)

        def interleave(i, carry):
            bv = bi_v[pl.ds(i * L, L)] + off
            dv = di_v[pl.ds(i * L, L)] + off
            posb = lane2 + i * (2 * L)
            plsc.store_scatter(gi_v, [posb], bv)
            plsc.store_scatter(gi_v, [posb + 1], dv)
            return carry

        lax.fori_loop(0, P // L, interleave, 0)

        def gather_group(g, carry):
            base = g * (FIRE * CHUNK)
            copies = []
            for k in range(FIRE):
                o = base + k * CHUNK
                copies.append(pltpu.async_copy(
                    xf.at[gi_v.at[pl.ds(o, CHUNK)]],
                    vals_v.at[pl.ds(o, CHUNK)], sem))
            for c in copies:
                c.wait()
            return carry

        lax.fori_loop(0, RP // (FIRE * CHUNK), gather_group, 0)

        pltpu.sync_copy(vals_v, out.at[r])


_gather_rows = functools.partial(
    pl.kernel,
    out_type=jax.ShapeDtypeStruct((ROWS, RP), jnp.float32),
    mesh=plsc.VectorSubcoreMesh(core_axis_name="c", subcore_axis_name="s"),
    compiler_params=pltpu.CompilerParams(needs_layout_passes=False),
    scratch_types=[
        pltpu.VMEM((P,), jnp.int32),
        pltpu.VMEM((P,), jnp.int32),
        pltpu.VMEM((RP,), jnp.int32),
        pltpu.VMEM((RP,), jnp.float32),
        pltpu.SemaphoreType.DMA,
    ],
)(_sc_body)


@jax.jit
def kernel(X, birth_idx, death_idx):
    xf = X.reshape(-1)
    bidx = birth_idx.astype(jnp.int32).reshape(ROWS, P)
    didx = death_idx.astype(jnp.int32).reshape(ROWS, P)
    out = _gather_rows(xf, bidx, didx)
    return out.reshape(D, B, P, 2)


# speedup vs baseline: 1.1521x; 1.1521x over previous
import functools

import jax
import jax.numpy as jnp
from jax import lax
from jax.experimental import pallas as pl
from jax.experimental.pallas import tpu as pltpu
from jax.experimental.pallas import tpu_sc as plsc

B, H, W = 64, 512, 512
HW = H * W
D = 2
P = 4096
ROWS = D * B
RP = 2 * P
NC, NS = 2, 16
NW = NC * NS
ROWS_PER_TILE = ROWS // NW
L = 16
CHUNK = 1024
FIRE = 8


def _sc_body(xf, bidx, didx, out, bi_v, di_v, gi_v, vals_v, sem):
    cid = lax.axis_index("c")
    sid = lax.axis_index("s")
    wid = sid * NC + cid
    lane2 = 2 * lax.iota(jnp.int32, L)

    for j in range(ROWS_PER_TILE):
        r = wid * ROWS_PER_TILE + j
        off = lax.rem(r, B) * HW

        pltpu.sync_copy(bidx.at[r], bi_v)
        pltpu.sync_copy(didx.at[r], di_v)

        def interleave(i, carry):
            bv = bi_v[pl.ds(i * L, L)] + off
            dv = di_v[pl.ds(i * L, L)] + off
            posb = lane2 + i * (2 * L)
            plsc.store_scatter(gi_v, [posb], bv)
            plsc.store_scatter(gi_v, [posb + 1], dv)
            return carry

        lax.fori_loop(0, P // L, interleave, 0)

        def gather_group(g, carry):
            base = g * (FIRE * CHUNK)
            copies = []
            for k in range(FIRE):
                o = base + k * CHUNK
                copies.append(pltpu.async_copy(
                    xf.at[gi_v.at[pl.ds(o, CHUNK)]],
                    vals_v.at[pl.ds(o, CHUNK)], sem))
            for c in copies:
                c.wait()
            return carry

        lax.fori_loop(0, RP // (FIRE * CHUNK), gather_group, 0)

        pltpu.sync_copy(vals_v, out.at[r])


_gather_rows = functools.partial(
    pl.kernel,
    out_type=jax.ShapeDtypeStruct((ROWS, RP), jnp.float32),
    mesh=plsc.VectorSubcoreMesh(core_axis_name="c", subcore_axis_name="s"),
    compiler_params=pltpu.CompilerParams(needs_layout_passes=False),
    scratch_types=[
        pltpu.VMEM((P,), jnp.int32),
        pltpu.VMEM((P,), jnp.int32),
        pltpu.VMEM((RP,), jnp.int32),
        pltpu.VMEM((RP,), jnp.float32),
        pltpu.SemaphoreType.DMA,
    ],
)(_sc_body)


@jax.jit
def kernel(X, birth_idx, death_idx):
    xf = X.reshape(-1)
    bidx = birth_idx.astype(jnp.int32).reshape(ROWS, P)
    didx = death_idx.astype(jnp.int32).reshape(ROWS, P)
    out = _gather_rows(xf, bidx, didx)
    return out.reshape(D, B, P, 2)


# speedup vs baseline: 1.2291x; 1.0669x over previous
import functools

import jax
import jax.numpy as jnp
from jax import lax
from jax.experimental import pallas as pl
from jax.experimental.pallas import tpu as pltpu
from jax.experimental.pallas import tpu_sc as plsc

B, H, W = 64, 512, 512
HW = H * W
D = 2
P = 4096
ROWS = D * B
RP = 2 * P
NC, NS = 2, 16
NW = NC * NS
RPT = ROWS // NW
L = 16
CHUNK = 1024
FIRE = RP // CHUNK


def _sc_body(xf, bidx, didx, out,
             bi0, bi1, di0, di1, gi0, gi1, va0, va1,
             sem_idx, sem_g, sem_out):
    cid = lax.axis_index("c")
    sid = lax.axis_index("s")
    wid = sid * NC + cid
    lane2 = 2 * lax.iota(jnp.int32, L)

    bi = (bi0, bi1)
    di = (di0, di1)
    gi = (gi0, gi1)
    va = (va0, va1)

    def start_idx(j):
        r = wid * RPT + j
        pltpu.async_copy(bidx.at[r], bi[j % 2], sem_idx)
        pltpu.async_copy(didx.at[r], di[j % 2], sem_idx)

    def wait_idx(j):
        pltpu.make_async_copy(bidx.at[0], bi[j % 2], sem_idx).wait()
        pltpu.make_async_copy(didx.at[0], di[j % 2], sem_idx).wait()

    def interleave(j):
        r = wid * RPT + j
        off = lax.rem(r, B) * HW
        bi_v, di_v, gi_v = bi[j % 2], di[j % 2], gi[j % 2]

        def body(i, carry):
            bv = bi_v[pl.ds(i * L, L)] + off
            dv = di_v[pl.ds(i * L, L)] + off
            posb = lane2 + i * (2 * L)
            plsc.store_scatter(gi_v, [posb], bv)
            plsc.store_scatter(gi_v, [posb + 1], dv)
            return carry

        lax.fori_loop(0, P // L, body, 0)

    def fire_gather(j):
        gi_v, va_v = gi[j % 2], va[j % 2]
        for k in range(FIRE):
            o = k * CHUNK
            pltpu.async_copy(
                xf.at[gi_v.at[pl.ds(o, CHUNK)]],
                va_v.at[pl.ds(o, CHUNK)], sem_g)

    def drain_gather(j):
        gi_v, va_v = gi[j % 2], va[j % 2]
        for k in range(FIRE):
            o = k * CHUNK
            pltpu.make_async_copy(
                xf.at[gi_v.at[pl.ds(o, CHUNK)]],
                va_v.at[pl.ds(o, CHUNK)], sem_g).wait()

    def start_out(j):
        r = wid * RPT + j
        pltpu.async_copy(va[j % 2], out.at[r], sem_out)

    def wait_out(j):
        r = wid * RPT + j
        pltpu.make_async_copy(va[j % 2], out.at[r], sem_out).wait()

    start_idx(0)
    wait_idx(0)
    start_idx(1)
    interleave(0)
    fire_gather(0)
    for j in range(1, RPT):
        wait_idx(j)
        if j + 1 < RPT:
            start_idx(j + 1)
        interleave(j)
        drain_gather(j - 1)
        if j >= 2:
            wait_out(j - 2)
        start_out(j - 1)
        fire_gather(j)
    drain_gather(RPT - 1)
    wait_out(RPT - 2)
    start_out(RPT - 1)
    wait_out(RPT - 1)


_gather_rows = functools.partial(
    pl.kernel,
    out_type=jax.ShapeDtypeStruct((ROWS, RP), jnp.float32),
    mesh=plsc.VectorSubcoreMesh(core_axis_name="c", subcore_axis_name="s"),
    compiler_params=pltpu.CompilerParams(needs_layout_passes=False),
    scratch_types=[
        pltpu.VMEM((P,), jnp.int32),
        pltpu.VMEM((P,), jnp.int32),
        pltpu.VMEM((P,), jnp.int32),
        pltpu.VMEM((P,), jnp.int32),
        pltpu.VMEM((RP,), jnp.int32),
        pltpu.VMEM((RP,), jnp.int32),
        pltpu.VMEM((RP,), jnp.float32),
        pltpu.VMEM((RP,), jnp.float32),
        pltpu.SemaphoreType.DMA,
        pltpu.SemaphoreType.DMA,
        pltpu.SemaphoreType.DMA,
    ],
)(_sc_body)


@jax.jit
def kernel(X, birth_idx, death_idx):
    xf = X.reshape(-1)
    bidx = birth_idx.astype(jnp.int32).reshape(ROWS, P)
    didx = death_idx.astype(jnp.int32).reshape(ROWS, P)
    out = _gather_rows(xf, bidx, didx)
    return out.reshape(D, B, P, 2)


# speedup vs baseline: 1.3226x; 1.0760x over previous
import functools

import jax
import jax.numpy as jnp
from jax import lax
from jax.experimental import pallas as pl
from jax.experimental.pallas import tpu as pltpu
from jax.experimental.pallas import tpu_sc as plsc

B, H, W = 64, 512, 512
HW = H * W
D = 2
P = 4096
ROWS = D * B
RP = 2 * P
NC, NS = 2, 16
NW = NC * NS
RPT = ROWS // NW
L = 16
CHUNK = 1024
FIRE = RP // CHUNK


def _sc_body(xf, bidx, didx, out,
             bi0, bi1, di0, di1, gi0, gi1, va0, va1,
             sem_idx, sem_g, sem_out):
    cid = lax.axis_index("c")
    sid = lax.axis_index("s")
    wid = sid * NC + cid

    bi = (bi0, bi1)
    di = (di0, di1)
    gi = (gi0, gi1)
    va = (va0, va1)

    def start_idx(j):
        r = wid * RPT + j
        pltpu.async_copy(bidx.at[r], bi[j % 2], sem_idx)
        pltpu.async_copy(didx.at[r], di[j % 2], sem_idx)

    def wait_idx(j):
        pltpu.make_async_copy(bidx.at[0], bi[j % 2], sem_idx).wait()
        pltpu.make_async_copy(didx.at[0], di[j % 2], sem_idx).wait()

    def reorder(j):
        r = wid * RPT + j
        off = lax.rem(r, B) * HW
        bi_v, di_v, gi_v = bi[j % 2], di[j % 2], gi[j % 2]

        def body(i, carry):
            t = lax.shift_right_logical(i, 3)
            co = (i & 7) * L
            bpos = (2 * (t & 3)) * 1024 + lax.shift_right_logical(t, 2) * 128 + co
            gi_v[pl.ds(bpos, L)] = bi_v[pl.ds(i * L, L)] + off
            gi_v[pl.ds(bpos + 1024, L)] = di_v[pl.ds(i * L, L)] + off
            return carry

        lax.fori_loop(0, P // L, body, 0)

    def fire_gather(j):
        gi_v, va_v = gi[j % 2], va[j % 2]
        for k in range(FIRE):
            o = k * CHUNK
            pltpu.async_copy(
                xf.at[gi_v.at[pl.ds(o, CHUNK)]],
                va_v.at[pl.ds(o, CHUNK)], sem_g)

    def drain_gather(j):
        gi_v, va_v = gi[j % 2], va[j % 2]
        for k in range(FIRE):
            o = k * CHUNK
            pltpu.make_async_copy(
                xf.at[gi_v.at[pl.ds(o, CHUNK)]],
                va_v.at[pl.ds(o, CHUNK)], sem_g).wait()

    def out_rows(j):
        r = wid * RPT + j
        g = lax.shift_right_logical(r, 3)
        rl = r & 7
        va_v = va[j % 2]
        for k in range(8):
            yield (va_v.at[pl.ds(k * 1024, 1024)],
                   out.at[g * 8 + k, pl.ds(rl * 1024, 1024)])

    def start_out(j):
        for src_ref, dst_ref in out_rows(j):
            pltpu.async_copy(src_ref, dst_ref, sem_out)

    def wait_out(j):
        for src_ref, dst_ref in out_rows(j):
            pltpu.make_async_copy(src_ref, dst_ref, sem_out).wait()

    start_idx(0)
    wait_idx(0)
    start_idx(1)
    reorder(0)
    fire_gather(0)
    for j in range(1, RPT):
        wait_idx(j)
        if j + 1 < RPT:
            start_idx(j + 1)
        reorder(j)
        drain_gather(j - 1)
        if j >= 2:
            wait_out(j - 2)
        start_out(j - 1)
        fire_gather(j)
    drain_gather(RPT - 1)
    wait_out(RPT - 2)
    start_out(RPT - 1)
    wait_out(RPT - 1)


_gather_rows = functools.partial(
    pl.kernel,
    out_type=jax.ShapeDtypeStruct((ROWS, RP), jnp.float32),
    mesh=plsc.VectorSubcoreMesh(core_axis_name="c", subcore_axis_name="s"),
    compiler_params=pltpu.CompilerParams(needs_layout_passes=False),
    scratch_types=[
        pltpu.VMEM((P,), jnp.int32),
        pltpu.VMEM((P,), jnp.int32),
        pltpu.VMEM((P,), jnp.int32),
        pltpu.VMEM((P,), jnp.int32),
        pltpu.VMEM((RP,), jnp.int32),
        pltpu.VMEM((RP,), jnp.int32),
        pltpu.VMEM((RP,), jnp.float32),
        pltpu.VMEM((RP,), jnp.float32),
        pltpu.SemaphoreType.DMA,
        pltpu.SemaphoreType.DMA,
        pltpu.SemaphoreType.DMA,
    ],
)(_sc_body)


@jax.jit
def kernel(X, birth_idx, death_idx):
    xf = X.reshape(-1)
    bidx = birth_idx.astype(jnp.int32).reshape(ROWS, P)
    didx = death_idx.astype(jnp.int32).reshape(ROWS, P)
    out = _gather_rows(xf, bidx, didx)
    out7 = out.reshape(D, 8, 4, 2, 8, 8, 128)
    return jnp.transpose(out7, (0, 1, 4, 5, 2, 6, 3)).reshape(D, B, P, 2)


# speedup vs baseline: 1.5107x; 1.1423x over previous
import functools

import jax
import jax.numpy as jnp
from jax import lax
from jax.experimental import pallas as pl
from jax.experimental.pallas import tpu as pltpu
from jax.experimental.pallas import tpu_sc as plsc

B, H, W = 64, 512, 512
HW = H * W
D = 2
P = 4096
ROWS = D * B
RP = 2 * P
NC, NS = 2, 16
NW = NC * NS
RPT = ROWS // NW
L = 16
CHUNK = 1024
FIRE = RP // CHUNK


def _sc_body(xf, bidx, didx, out,
             bi0, bi1, di0, di1, gi0, gi1, va0, va1,
             sem_idx, sem_g, sem_out):
    cid = lax.axis_index("c")
    sid = lax.axis_index("s")
    wid = sid * NC + cid

    bi = (bi0, bi1)
    di = (di0, di1)
    gi = (gi0, gi1)
    va = (va0, va1)

    def start_idx(j):
        r = wid * RPT + j
        pltpu.async_copy(bidx.at[r], bi[j % 2], sem_idx)
        pltpu.async_copy(didx.at[r], di[j % 2], sem_idx)

    def wait_idx(j):
        pltpu.make_async_copy(bidx.at[0], bi[j % 2], sem_idx).wait()
        pltpu.make_async_copy(didx.at[0], di[j % 2], sem_idx).wait()

    def reorder(j):
        r = wid * RPT + j
        off = lax.rem(r, B) * HW
        bi_v, di_v, gi_v = bi[j % 2], di[j % 2], gi[j % 2]

        def body(i, carry):
            bpos = lax.shift_right_logical(i, 3) * 256 + (i & 7) * L
            gi_v[pl.ds(bpos, L)] = bi_v[pl.ds(i * L, L)] + off
            gi_v[pl.ds(bpos + 128, L)] = di_v[pl.ds(i * L, L)] + off
            return carry

        lax.fori_loop(0, P // L, body, 0)

    def fire_gather(j):
        gi_v, va_v = gi[j % 2], va[j % 2]
        for k in range(FIRE):
            o = k * CHUNK
            pltpu.async_copy(
                xf.at[gi_v.at[pl.ds(o, CHUNK)]],
                va_v.at[pl.ds(o, CHUNK)], sem_g)

    def drain_gather(j):
        gi_v, va_v = gi[j % 2], va[j % 2]
        for k in range(FIRE):
            o = k * CHUNK
            pltpu.make_async_copy(
                xf.at[gi_v.at[pl.ds(o, CHUNK)]],
                va_v.at[pl.ds(o, CHUNK)], sem_g).wait()

    def start_out(j):
        r = wid * RPT + j
        pltpu.async_copy(va[j % 2], out.at[pl.ds(r * RP, RP)], sem_out)

    def wait_out(j):
        r = wid * RPT + j
        pltpu.make_async_copy(va[j % 2], out.at[pl.ds(r * RP, RP)], sem_out).wait()

    start_idx(0)
    wait_idx(0)
    start_idx(1)
    reorder(0)
    fire_gather(0)
    for j in range(1, RPT):
        wait_idx(j)
        if j + 1 < RPT:
            start_idx(j + 1)
        reorder(j)
        drain_gather(j - 1)
        if j >= 2:
            wait_out(j - 2)
        start_out(j - 1)
        fire_gather(j)
    drain_gather(RPT - 1)
    wait_out(RPT - 2)
    start_out(RPT - 1)
    wait_out(RPT - 1)


_gather_rows = functools.partial(
    pl.kernel,
    out_type=jax.ShapeDtypeStruct((ROWS * RP,), jnp.float32),
    mesh=plsc.VectorSubcoreMesh(core_axis_name="c", subcore_axis_name="s"),
    compiler_params=pltpu.CompilerParams(needs_layout_passes=False),
    scratch_types=[
        pltpu.VMEM((P,), jnp.int32),
        pltpu.VMEM((P,), jnp.int32),
        pltpu.VMEM((P,), jnp.int32),
        pltpu.VMEM((P,), jnp.int32),
        pltpu.VMEM((RP,), jnp.int32),
        pltpu.VMEM((RP,), jnp.int32),
        pltpu.VMEM((RP,), jnp.float32),
        pltpu.VMEM((RP,), jnp.float32),
        pltpu.SemaphoreType.DMA,
        pltpu.SemaphoreType.DMA,
        pltpu.SemaphoreType.DMA,
    ],
)(_sc_body)


@jax.jit
def kernel(X, birth_idx, death_idx):
    xf = X.reshape(-1)
    bidx = birth_idx.astype(jnp.int32).reshape(ROWS, P)
    didx = death_idx.astype(jnp.int32).reshape(ROWS, P)
    out = _gather_rows(xf, bidx, didx)
    out5 = out.reshape(D, B, P // 128, 2, 128)
    return jnp.transpose(out5, (0, 1, 2, 4, 3)).reshape(D, B, P, 2)


# speedup vs baseline: 1.5150x; 1.0028x over previous
import functools

import jax
import jax.numpy as jnp
from jax import lax
from jax.experimental import pallas as pl
from jax.experimental.pallas import tpu as pltpu
from jax.experimental.pallas import tpu_sc as plsc

B, H, W = 64, 512, 512
HW = H * W
D = 2
P = 4096
ROWS = D * B
RP = 2 * P
NC, NS = 2, 16
NW = NC * NS
RPT = ROWS // NW
L = 16
CHUNK = 2048
FIRE = RP // CHUNK


def _sc_body(xf, bidx, didx, out,
             bi0, bi1, di0, di1, gi0, gi1, va0, va1,
             sem_idx, sem_g, sem_out):
    cid = lax.axis_index("c")
    sid = lax.axis_index("s")
    wid = sid * NC + cid

    bi = (bi0, bi1)
    di = (di0, di1)
    gi = (gi0, gi1)
    va = (va0, va1)

    def start_idx(j):
        r = wid * RPT + j
        pltpu.async_copy(bidx.at[r], bi[j % 2], sem_idx)
        pltpu.async_copy(didx.at[r], di[j % 2], sem_idx)

    def wait_idx(j):
        pltpu.make_async_copy(bidx.at[0], bi[j % 2], sem_idx).wait()
        pltpu.make_async_copy(didx.at[0], di[j % 2], sem_idx).wait()

    def reorder(j):
        r = wid * RPT + j
        off = lax.rem(r, B) * HW
        bi_v, di_v, gi_v = bi[j % 2], di[j % 2], gi[j % 2]

        def body(i, carry):
            bpos = lax.shift_right_logical(i, 3) * 256 + (i & 7) * L
            gi_v[pl.ds(bpos, L)] = bi_v[pl.ds(i * L, L)] + off
            gi_v[pl.ds(bpos + 128, L)] = di_v[pl.ds(i * L, L)] + off
            return carry

        lax.fori_loop(0, P // L, body, 0)

    def fire_gather(j):
        gi_v, va_v = gi[j % 2], va[j % 2]
        for k in range(FIRE):
            o = k * CHUNK
            pltpu.async_copy(
                xf.at[gi_v.at[pl.ds(o, CHUNK)]],
                va_v.at[pl.ds(o, CHUNK)], sem_g)

    def drain_gather(j):
        gi_v, va_v = gi[j % 2], va[j % 2]
        for k in range(FIRE):
            o = k * CHUNK
            pltpu.make_async_copy(
                xf.at[gi_v.at[pl.ds(o, CHUNK)]],
                va_v.at[pl.ds(o, CHUNK)], sem_g).wait()

    def start_out(j):
        r = wid * RPT + j
        pltpu.async_copy(va[j % 2], out.at[pl.ds(r * RP, RP)], sem_out)

    def wait_out(j):
        r = wid * RPT + j
        pltpu.make_async_copy(va[j % 2], out.at[pl.ds(r * RP, RP)], sem_out).wait()

    start_idx(0)
    wait_idx(0)
    start_idx(1)
    reorder(0)
    fire_gather(0)
    for j in range(1, RPT):
        wait_idx(j)
        if j + 1 < RPT:
            start_idx(j + 1)
        reorder(j)
        drain_gather(j - 1)
        if j >= 2:
            wait_out(j - 2)
        start_out(j - 1)
        fire_gather(j)
    drain_gather(RPT - 1)
    wait_out(RPT - 2)
    start_out(RPT - 1)
    wait_out(RPT - 1)


_gather_rows = functools.partial(
    pl.kernel,
    out_type=jax.ShapeDtypeStruct((ROWS * RP,), jnp.float32),
    mesh=plsc.VectorSubcoreMesh(core_axis_name="c", subcore_axis_name="s"),
    compiler_params=pltpu.CompilerParams(needs_layout_passes=False),
    scratch_types=[
        pltpu.VMEM((P,), jnp.int32),
        pltpu.VMEM((P,), jnp.int32),
        pltpu.VMEM((P,), jnp.int32),
        pltpu.VMEM((P,), jnp.int32),
        pltpu.VMEM((RP,), jnp.int32),
        pltpu.VMEM((RP,), jnp.int32),
        pltpu.VMEM((RP,), jnp.float32),
        pltpu.VMEM((RP,), jnp.float32),
        pltpu.SemaphoreType.DMA,
        pltpu.SemaphoreType.DMA,
        pltpu.SemaphoreType.DMA,
    ],
)(_sc_body)


@jax.jit
def kernel(X, birth_idx, death_idx):
    xf = X.reshape(-1)
    bidx = birth_idx.astype(jnp.int32).reshape(ROWS, P)
    didx = death_idx.astype(jnp.int32).reshape(ROWS, P)
    out = _gather_rows(xf, bidx, didx)
    out5 = out.reshape(D, B, P // 128, 2, 128)
    return jnp.transpose(out5, (0, 1, 2, 4, 3)).reshape(D, B, P, 2)
